# baseline (speedup 1.0000x reference)
import jax
import jax.numpy as jnp
from jax import lax
from jax.experimental import pallas as pl
from jax.experimental.pallas import tpu as pltpu
from jax.experimental.pallas import tpu_sc as plsc

D_MODEL = 1024
NUM_EXPERTS = 8
HIDDEN = 4096
TOP_K = 2
N_TOKENS = 8192
M_SLOTS = N_TOKENS * TOP_K
CAP = 1280
EC = NUM_EXPERTS * CAP

NW = 32
ROWS_PER_W = EC // NW
GCHUNK = 64
TOK_PER_W = N_TOKENS // NW
CCHUNK = 16

RB = 256
HB = 512
N_RB = CAP // RB
N_HB = HIDDEN // HB

_INV_SQRT2 = 0.7071067811865476


def _router_body(x_ref, wr_ref, br_ref, e1_ref, e2_ref, g1_ref, g2_ref, imp_ref):
    xb = x_ref[...]
    logits = jnp.dot(xb, wr_ref[...], preferred_element_type=jnp.float32)
    logits = logits + br_ref[0, :][None, :]
    mx = jnp.max(logits, axis=-1, keepdims=True)
    p = jnp.exp(logits - mx)
    s = jnp.sum(p, axis=-1, keepdims=True)
    probs = p / s
    imp_ref[0, 0, :] = jnp.sum(probs, axis=0)

    iota8 = lax.broadcasted_iota(jnp.int32, probs.shape, 1)
    m1 = jnp.max(probs, axis=-1, keepdims=True)
    i1 = jnp.min(jnp.where(probs == m1, iota8, NUM_EXPERTS), axis=-1)
    mask1 = iota8 == i1[:, None]
    probs2 = jnp.where(mask1, -jnp.inf, probs)
    m2 = jnp.max(probs2, axis=-1, keepdims=True)
    i2 = jnp.min(jnp.where(probs2 == m2, iota8, NUM_EXPERTS), axis=-1)
    gs = m1[:, 0] + m2[:, 0]
    e1_ref[...] = i1
    e2_ref[...] = i2
    g1_ref[...] = m1[:, 0] / gs
    g2_ref[...] = m2[:, 0] / gs


def _router(x2, Wr, br2):
    nblk = N_TOKENS // 1024
    return pl.pallas_call(
        _router_body,
        grid=(nblk,),
        in_specs=[
            pl.BlockSpec((1024, D_MODEL), lambda b: (b, 0)),
            pl.BlockSpec((D_MODEL, NUM_EXPERTS), lambda b: (0, 0)),
            pl.BlockSpec((1, NUM_EXPERTS), lambda b: (0, 0)),
        ],
        out_specs=[
            pl.BlockSpec((1024,), lambda b: (b,)),
            pl.BlockSpec((1024,), lambda b: (b,)),
            pl.BlockSpec((1024,), lambda b: (b,)),
            pl.BlockSpec((1024,), lambda b: (b,)),
            pl.BlockSpec((1, 1, NUM_EXPERTS), lambda b: (b, 0, 0)),
        ],
        out_shape=[
            jax.ShapeDtypeStruct((N_TOKENS,), jnp.int32),
            jax.ShapeDtypeStruct((N_TOKENS,), jnp.int32),
            jax.ShapeDtypeStruct((N_TOKENS,), jnp.float32),
            jax.ShapeDtypeStruct((N_TOKENS,), jnp.float32),
            jax.ShapeDtypeStruct((nblk, 1, NUM_EXPERTS), jnp.float32),
        ],
    )(x2, Wr, br2)


def _dispatch_body(fi_ref, gf_ref, imp_ref, asc_ref, acb_ref, gq_ref, aux_ref):
    fi = fi_ref[...]
    gf = gf_ref[...]
    r_i = lax.broadcasted_iota(jnp.int32, (128, 128), 0)
    c_i = lax.broadcasted_iota(jnp.int32, (128, 128), 1)
    U = (r_i < c_i).astype(jnp.float32)
    L = (r_i > c_i).astype(jnp.float32)

    rank = jnp.zeros((128, 128), jnp.float32)
    totals = []
    for e in range(NUM_EXPERTS):
        me = (fi == e).astype(jnp.float32)
        pe = jnp.dot(me, U, preferred_element_type=jnp.float32)
        se = jnp.sum(me, axis=-1, keepdims=True)
        oe = jnp.dot(L, se, preferred_element_type=jnp.float32)
        rank = rank + (pe + oe) * me
        totals.append(jnp.sum(se))
    rank = rank.astype(jnp.int32)
    valid = rank < CAP
    addr = fi * CAP + rank
    asc_ref[...] = jnp.where(valid, addr, jnp.int32(1 << 20))
    acb_ref[...] = jnp.where(valid, addr, 0)
    gq_ref[...] = jnp.where(valid, gf, 0.0)

    imp = jnp.sum(imp_ref[...], axis=0)
    load = jnp.minimum(jnp.stack(totals), float(CAP))
    impn = imp / jnp.sum(imp)
    loadn = load / jnp.sum(load)
    aux = jnp.sum(impn * loadn) * float(NUM_EXPERTS * NUM_EXPERTS)
    aux_ref[...] = jnp.broadcast_to(aux, (1, 1))


def _dispatch(fi2, gf2, imp8):
    return pl.pallas_call(
        _dispatch_body,
        out_shape=[
            jax.ShapeDtypeStruct((128, 128), jnp.int32),
            jax.ShapeDtypeStruct((128, 128), jnp.int32),
            jax.ShapeDtypeStruct((128, 128), jnp.float32),
            jax.ShapeDtypeStruct((1, 1), jnp.float32),
        ],
    )(fi2, gf2, imp8)


def _sc_mesh():
    return plsc.VectorSubcoreMesh(core_axis_name="c", subcore_axis_name="s")


def _scgather_body(asc_hbm, x_hbm, xe_hbm, asc_v, seg_v, rows_v, sem):
    wid = lax.axis_index("s") * 2 + lax.axis_index("c")
    lo = wid * ROWS_PER_W
    pltpu.sync_copy(asc_hbm, asc_v)
    zero = jnp.zeros((16,), jnp.int32)
    for j in range(ROWS_PER_W // 16):
        seg_v[pl.ds(j * 16, 16)] = zero

    iota = lax.iota(jnp.int32, 16)

    def step(i, carry):
        a = asc_v[pl.ds(i * 16, 16)]
        la = a - lo
        mask = (a >= lo) & (a < lo + ROWS_PER_W)
        val = lax.shift_right_logical(i * 16 + iota, 1)
        plsc.store_scatter(seg_v, [jnp.where(mask, la, 0)], val, mask=mask)
        return carry

    lax.fori_loop(0, M_SLOTS // 16, step, 0)

    for ch in range(ROWS_PER_W // GCHUNK):
        pltpu.async_copy(
            x_hbm.at[seg_v.at[pl.ds(ch * GCHUNK, GCHUNK)]], rows_v, sem
        ).wait()
        pltpu.sync_copy(rows_v, xe_hbm.at[pl.ds(lo + ch * GCHUNK, GCHUNK)])


def _scgather(a_sc, x2):
    k = pl.kernel(
        _scgather_body,
        out_type=jax.ShapeDtypeStruct((EC, D_MODEL), jnp.float32),
        mesh=_sc_mesh(),
        compiler_params=pltpu.CompilerParams(needs_layout_passes=False),
        scratch_types=[
            pltpu.VMEM((M_SLOTS,), jnp.int32),
            pltpu.VMEM((ROWS_PER_W,), jnp.int32),
            pltpu.VMEM((GCHUNK, D_MODEL), jnp.float32),
            pltpu.SemaphoreType.DMA,
        ],
    )
    return k(a_sc, x2)


def _ffn_body(x_ref, w1_ref, b1_ref, w2_ref, b2_ref, o_ref):
    h = pl.program_id(2)
    xb = x_ref[...]
    hb = jnp.dot(xb, w1_ref[0], preferred_element_type=jnp.float32) + b1_ref[0]
    hb = 0.5 * hb * (1.0 + lax.erf(hb * _INV_SQRT2))
    contrib = jnp.dot(hb, w2_ref[0], preferred_element_type=jnp.float32)

    @pl.when(h == 0)
    def _():
        o_ref[...] = contrib + b2_ref[0]

    @pl.when(h > 0)
    def _():
        o_ref[...] = o_ref[...] + contrib


def _ffn(xe, W1, b1, W2, b2):
    return pl.pallas_call(
        _ffn_body,
        grid=(NUM_EXPERTS, N_RB, N_HB),
        in_specs=[
            pl.BlockSpec((RB, D_MODEL), lambda e, r, h: (e * N_RB + r, 0)),
            pl.BlockSpec((1, D_MODEL, HB), lambda e, r, h: (e, 0, h)),
            pl.BlockSpec((1, 1, HB), lambda e, r, h: (e, 0, h)),
            pl.BlockSpec((1, HB, D_MODEL), lambda e, r, h: (e, h, 0)),
            pl.BlockSpec((1, 1, D_MODEL), lambda e, r, h: (e, 0, 0)),
        ],
        out_specs=pl.BlockSpec((RB, D_MODEL), lambda e, r, h: (e * N_RB + r, 0)),
        out_shape=jax.ShapeDtypeStruct((EC, D_MODEL), jnp.float32),
        compiler_params=pltpu.CompilerParams(
            dimension_semantics=("parallel", "parallel", "arbitrary"),
        ),
    )(xe, W1, b1.reshape(NUM_EXPERTS, 1, HIDDEN), W2,
      b2.reshape(NUM_EXPERTS, 1, D_MODEL))


def _combine_body(ob_hbm, acb_hbm, gq_hbm, y_hbm, acb_v, gq_v, idx_v,
                  rows_v, y_v, gsem, osem):
    wid = lax.axis_index("s") * 2 + lax.axis_index("c")
    tbase = wid * TOK_PER_W
    sbase = tbase * 2
    pltpu.sync_copy(acb_hbm.at[pl.ds(sbase, TOK_PER_W * 2)], acb_v)
    pltpu.sync_copy(gq_hbm.at[pl.ds(sbase, TOK_PER_W * 2)], gq_v)
    iota = lax.iota(jnp.int32, 16)
    nch = TOK_PER_W // CCHUNK

    def fire(c, p):
        sidx = 2 * (c * CCHUNK + iota)
        idx_v[pl.ds(p * 2 * CCHUNK, 16)] = plsc.load_gather(acb_v, [sidx])
        idx_v[pl.ds(p * 2 * CCHUNK + 16, 16)] = plsc.load_gather(acb_v, [sidx + 1])
        return pltpu.async_copy(
            ob_hbm.at[idx_v.at[pl.ds(p * 2 * CCHUNK, 2 * CCHUNK)]],
            rows_v.at[p], gsem,
        )

    cps = {0: fire(0, 0)}
    ocs = {}
    for c in range(nch):
        p = c & 1
        if c + 1 < nch:
            cps[(c + 1) & 1] = fire(c + 1, (c + 1) & 1)
        cps[p].wait()
        if c >= 2:
            ocs[p].wait()

        def tok(t, carry):
            st = 2 * (c * CCHUNK + t)
            ga = plsc.load_gather(gq_v, [jnp.full((16,), st, jnp.int32)])
            gb = plsc.load_gather(gq_v, [jnp.full((16,), st + 1, jnp.int32)])
            for j in range(D_MODEL // 16):
                y_v[p, t, pl.ds(j * 16, 16)] = (
                    ga * rows_v[p, t, pl.ds(j * 16, 16)]
                    + gb * rows_v[p, CCHUNK + t, pl.ds(j * 16, 16)]
                )
            return carry

        lax.fori_loop(0, CCHUNK, tok, 0)
        ocs[p] = pltpu.async_copy(
            y_v.at[p], y_hbm.at[pl.ds(tbase + c * CCHUNK, CCHUNK)], osem
        )
    ocs[(nch - 2) & 1].wait()
    ocs[(nch - 1) & 1].wait()


def _combine(outbuf, a_cb, gq):
    k = pl.kernel(
        _combine_body,
        out_type=jax.ShapeDtypeStruct((N_TOKENS, D_MODEL), jnp.float32),
        mesh=_sc_mesh(),
        compiler_params=pltpu.CompilerParams(needs_layout_passes=False),
        scratch_types=[
            pltpu.VMEM((TOK_PER_W * 2,), jnp.int32),
            pltpu.VMEM((TOK_PER_W * 2,), jnp.float32),
            pltpu.VMEM((2 * 2 * CCHUNK,), jnp.int32),
            pltpu.VMEM((2, 2 * CCHUNK, D_MODEL), jnp.float32),
            pltpu.VMEM((2, CCHUNK, D_MODEL), jnp.float32),
            pltpu.SemaphoreType.DMA,
            pltpu.SemaphoreType.DMA,
        ],
    )
    return k(outbuf, a_cb, gq)


def kernel(x, Wr, br, W1, b1, W2, b2):
    B, T, C = x.shape
    x2 = x.reshape(B * T, C)
    e1, e2, g1, g2, imp = _router(x2, Wr, br.reshape(1, NUM_EXPERTS))
    fi2 = jnp.stack([e1, e2], axis=-1).reshape(128, 128)
    gf2 = jnp.stack([g1, g2], axis=-1).reshape(128, 128)
    imp8 = imp.reshape(N_TOKENS // 1024, NUM_EXPERTS)
    a_sc2, a_cb2, gq2, aux = _dispatch(fi2, gf2, imp8)
    xe = _scgather(a_sc2.reshape(M_SLOTS), x2)
    outbuf = _ffn(xe, W1, b1, W2, b2)
    y = _combine(outbuf, a_cb2.reshape(M_SLOTS), gq2.reshape(M_SLOTS))
    return y.reshape(B, T, C), aux[0, 0]


# speedup vs baseline: 1.4601x; 1.4601x over previous
import jax
import jax.numpy as jnp
from jax import lax
from jax.experimental import pallas as pl
from jax.experimental.pallas import tpu as pltpu
from jax.experimental.pallas import tpu_sc as plsc

D_MODEL = 1024
NUM_EXPERTS = 8
HIDDEN = 4096
TOP_K = 2
N_TOKENS = 8192
M_SLOTS = N_TOKENS * TOP_K
CAP = 1280
EC = NUM_EXPERTS * CAP

NW = 32
ROWS_PER_W = EC // NW
GCHUNK = 64
TOK_PER_W = N_TOKENS // NW
CCHUNK = 16

RB = 256
HB = 512
N_RB = CAP // RB
N_HB = HIDDEN // HB

_INV_SQRT2 = 0.7071067811865476


def _router_body(x_ref, wr_ref, br_ref, e1_ref, e2_ref, g1_ref, g2_ref, imp_ref):
    xb = x_ref[...]
    logits = jnp.dot(xb, wr_ref[...], preferred_element_type=jnp.float32)
    logits = logits + br_ref[0, :][None, :]
    mx = jnp.max(logits, axis=-1, keepdims=True)
    p = jnp.exp(logits - mx)
    s = jnp.sum(p, axis=-1, keepdims=True)
    probs = p / s
    imp_ref[0, 0, :] = jnp.sum(probs, axis=0)

    iota8 = lax.broadcasted_iota(jnp.int32, probs.shape, 1)
    m1 = jnp.max(probs, axis=-1, keepdims=True)
    i1 = jnp.min(jnp.where(probs == m1, iota8, NUM_EXPERTS), axis=-1)
    mask1 = iota8 == i1[:, None]
    probs2 = jnp.where(mask1, -jnp.inf, probs)
    m2 = jnp.max(probs2, axis=-1, keepdims=True)
    i2 = jnp.min(jnp.where(probs2 == m2, iota8, NUM_EXPERTS), axis=-1)
    gs = m1[:, 0] + m2[:, 0]
    e1_ref[...] = i1
    e2_ref[...] = i2
    g1_ref[...] = m1[:, 0] / gs
    g2_ref[...] = m2[:, 0] / gs


def _router(x2, Wr, br2):
    nblk = N_TOKENS // 1024
    return pl.pallas_call(
        _router_body,
        grid=(nblk,),
        in_specs=[
            pl.BlockSpec((1024, D_MODEL), lambda b: (b, 0)),
            pl.BlockSpec((D_MODEL, NUM_EXPERTS), lambda b: (0, 0)),
            pl.BlockSpec((1, NUM_EXPERTS), lambda b: (0, 0)),
        ],
        out_specs=[
            pl.BlockSpec((1024,), lambda b: (b,)),
            pl.BlockSpec((1024,), lambda b: (b,)),
            pl.BlockSpec((1024,), lambda b: (b,)),
            pl.BlockSpec((1024,), lambda b: (b,)),
            pl.BlockSpec((1, 1, NUM_EXPERTS), lambda b: (b, 0, 0)),
        ],
        out_shape=[
            jax.ShapeDtypeStruct((N_TOKENS,), jnp.int32),
            jax.ShapeDtypeStruct((N_TOKENS,), jnp.int32),
            jax.ShapeDtypeStruct((N_TOKENS,), jnp.float32),
            jax.ShapeDtypeStruct((N_TOKENS,), jnp.float32),
            jax.ShapeDtypeStruct((nblk, 1, NUM_EXPERTS), jnp.float32),
        ],
    )(x2, Wr, br2)


def _dispatch_body(fi_ref, gf_ref, imp_ref, asc_ref, acb_ref, gq_ref, aux_ref):
    fi = fi_ref[...]
    gf = gf_ref[...]
    r_i = lax.broadcasted_iota(jnp.int32, (128, 128), 0)
    c_i = lax.broadcasted_iota(jnp.int32, (128, 128), 1)
    U = (r_i < c_i).astype(jnp.float32)
    L = (r_i > c_i).astype(jnp.float32)

    rank = jnp.zeros((128, 128), jnp.float32)
    totals = []
    for e in range(NUM_EXPERTS):
        me = (fi == e).astype(jnp.float32)
        pe = jnp.dot(me, U, preferred_element_type=jnp.float32)
        se = jnp.sum(me, axis=-1, keepdims=True)
        oe = jnp.dot(L, se, preferred_element_type=jnp.float32)
        rank = rank + (pe + oe) * me
        totals.append(jnp.sum(se))
    rank = rank.astype(jnp.int32)
    valid = rank < CAP
    addr = fi * CAP + rank
    asc_ref[...] = jnp.where(valid, addr, jnp.int32(1 << 20))
    acb_ref[...] = jnp.where(valid, addr, 0)
    gq_ref[...] = jnp.where(valid, gf, 0.0)

    imp = jnp.sum(imp_ref[...], axis=0)
    load = jnp.minimum(jnp.stack(totals), float(CAP))
    impn = imp / jnp.sum(imp)
    loadn = load / jnp.sum(load)
    aux = jnp.sum(impn * loadn) * float(NUM_EXPERTS * NUM_EXPERTS)
    aux_ref[...] = jnp.broadcast_to(aux, (1, 1))


def _dispatch(fi2, gf2, imp8):
    return pl.pallas_call(
        _dispatch_body,
        out_shape=[
            jax.ShapeDtypeStruct((128, 128), jnp.int32),
            jax.ShapeDtypeStruct((128, 128), jnp.int32),
            jax.ShapeDtypeStruct((128, 128), jnp.float32),
            jax.ShapeDtypeStruct((1, 1), jnp.float32),
        ],
    )(fi2, gf2, imp8)


def _sc_mesh():
    return plsc.VectorSubcoreMesh(core_axis_name="c", subcore_axis_name="s")


def _scgather_body(asc_hbm, x_hbm, xe_hbm, asc_v, seg_v, rows_v, sem):
    wid = lax.axis_index("s") * 2 + lax.axis_index("c")
    lo = wid * ROWS_PER_W
    pltpu.sync_copy(asc_hbm, asc_v)
    zero = jnp.zeros((16,), jnp.int32)
    for j in range(ROWS_PER_W // 16):
        seg_v[pl.ds(j * 16, 16)] = zero

    iota = lax.iota(jnp.int32, 16)

    def step(i, carry):
        a = asc_v[pl.ds(i * 16, 16)]
        la = a - lo
        mask = (a >= lo) & (a < lo + ROWS_PER_W)
        val = lax.shift_right_logical(i * 16 + iota, 1)
        plsc.store_scatter(seg_v, [jnp.where(mask, la, 0)], val, mask=mask)
        return carry

    lax.fori_loop(0, M_SLOTS // 16, step, 0)

    for ch in range(ROWS_PER_W // GCHUNK):
        pltpu.async_copy(
            x_hbm.at[seg_v.at[pl.ds(ch * GCHUNK, GCHUNK)]], rows_v, sem
        ).wait()
        pltpu.sync_copy(rows_v, xe_hbm.at[pl.ds(lo + ch * GCHUNK, GCHUNK)])


def _scgather(a_sc, x2):
    k = pl.kernel(
        _scgather_body,
        out_type=jax.ShapeDtypeStruct((EC, D_MODEL), jnp.float32),
        mesh=_sc_mesh(),
        compiler_params=pltpu.CompilerParams(needs_layout_passes=False),
        scratch_types=[
            pltpu.VMEM((M_SLOTS,), jnp.int32),
            pltpu.VMEM((ROWS_PER_W,), jnp.int32),
            pltpu.VMEM((GCHUNK, D_MODEL), jnp.float32),
            pltpu.SemaphoreType.DMA,
        ],
    )
    return k(a_sc, x2)


def _ffn_body(x_ref, w1_ref, b1_ref, w2_ref, b2_ref, o_ref):
    h = pl.program_id(1)
    xb = x_ref[...].astype(jnp.bfloat16)
    hb = jnp.dot(xb, w1_ref[0].astype(jnp.bfloat16),
                 preferred_element_type=jnp.float32) + b1_ref[0]
    hb = 0.5 * hb * (1.0 + lax.erf(hb * _INV_SQRT2))
    contrib = jnp.dot(hb.astype(jnp.bfloat16), w2_ref[0].astype(jnp.bfloat16),
                      preferred_element_type=jnp.float32)

    @pl.when(h == 0)
    def _():
        o_ref[...] = contrib + b2_ref[0]

    @pl.when(h > 0)
    def _():
        o_ref[...] = o_ref[...] + contrib


def _ffn(xe, W1, b1, W2, b2):
    return pl.pallas_call(
        _ffn_body,
        grid=(NUM_EXPERTS, N_HB),
        in_specs=[
            pl.BlockSpec((CAP, D_MODEL), lambda e, h: (e, 0)),
            pl.BlockSpec((1, D_MODEL, HB), lambda e, h: (e, 0, h)),
            pl.BlockSpec((1, 1, HB), lambda e, h: (e, 0, h)),
            pl.BlockSpec((1, HB, D_MODEL), lambda e, h: (e, h, 0)),
            pl.BlockSpec((1, 1, D_MODEL), lambda e, h: (e, 0, 0)),
        ],
        out_specs=pl.BlockSpec((CAP, D_MODEL), lambda e, h: (e, 0)),
        out_shape=jax.ShapeDtypeStruct((EC, D_MODEL), jnp.float32),
        compiler_params=pltpu.CompilerParams(
            dimension_semantics=("parallel", "arbitrary"),
        ),
    )(xe, W1, b1.reshape(NUM_EXPERTS, 1, HIDDEN), W2,
      b2.reshape(NUM_EXPERTS, 1, D_MODEL))


def _combine_body(ob_hbm, acb_hbm, gq_hbm, y_hbm, acb_v, gq_v, idx_v,
                  rows_v, y_v, gsem, osem):
    wid = lax.axis_index("s") * 2 + lax.axis_index("c")
    tbase = wid * TOK_PER_W
    sbase = tbase * 2
    pltpu.sync_copy(acb_hbm.at[pl.ds(sbase, TOK_PER_W * 2)], acb_v)
    pltpu.sync_copy(gq_hbm.at[pl.ds(sbase, TOK_PER_W * 2)], gq_v)
    iota = lax.iota(jnp.int32, 16)
    nch = TOK_PER_W // CCHUNK

    def fire(c, p):
        sidx = 2 * (c * CCHUNK + iota)
        idx_v[pl.ds(p * 2 * CCHUNK, 16)] = plsc.load_gather(acb_v, [sidx])
        idx_v[pl.ds(p * 2 * CCHUNK + 16, 16)] = plsc.load_gather(acb_v, [sidx + 1])
        return pltpu.async_copy(
            ob_hbm.at[idx_v.at[pl.ds(p * 2 * CCHUNK, 2 * CCHUNK)]],
            rows_v.at[p], gsem,
        )

    cps = {0: fire(0, 0)}
    ocs = {}
    for c in range(nch):
        p = c & 1
        if c + 1 < nch:
            cps[(c + 1) & 1] = fire(c + 1, (c + 1) & 1)
        cps[p].wait()
        if c >= 2:
            ocs[p].wait()

        def tok(t, carry):
            st = 2 * (c * CCHUNK + t)
            ga = plsc.load_gather(gq_v, [jnp.full((16,), st, jnp.int32)])
            gb = plsc.load_gather(gq_v, [jnp.full((16,), st + 1, jnp.int32)])
            for j in range(D_MODEL // 16):
                y_v[p, t, pl.ds(j * 16, 16)] = (
                    ga * rows_v[p, t, pl.ds(j * 16, 16)]
                    + gb * rows_v[p, CCHUNK + t, pl.ds(j * 16, 16)]
                )
            return carry

        lax.fori_loop(0, CCHUNK, tok, 0)
        ocs[p] = pltpu.async_copy(
            y_v.at[p], y_hbm.at[pl.ds(tbase + c * CCHUNK, CCHUNK)], osem
        )
    ocs[(nch - 2) & 1].wait()
    ocs[(nch - 1) & 1].wait()


def _combine(outbuf, a_cb, gq):
    k = pl.kernel(
        _combine_body,
        out_type=jax.ShapeDtypeStruct((N_TOKENS, D_MODEL), jnp.float32),
        mesh=_sc_mesh(),
        compiler_params=pltpu.CompilerParams(needs_layout_passes=False),
        scratch_types=[
            pltpu.VMEM((TOK_PER_W * 2,), jnp.int32),
            pltpu.VMEM((TOK_PER_W * 2,), jnp.float32),
            pltpu.VMEM((2 * 2 * CCHUNK,), jnp.int32),
            pltpu.VMEM((2, 2 * CCHUNK, D_MODEL), jnp.float32),
            pltpu.VMEM((2, CCHUNK, D_MODEL), jnp.float32),
            pltpu.SemaphoreType.DMA,
            pltpu.SemaphoreType.DMA,
        ],
    )
    return k(outbuf, a_cb, gq)


def kernel(x, Wr, br, W1, b1, W2, b2):
    B, T, C = x.shape
    x2 = x.reshape(B * T, C)
    e1, e2, g1, g2, imp = _router(x2, Wr, br.reshape(1, NUM_EXPERTS))
    fi2 = jnp.stack([e1, e2], axis=-1).reshape(128, 128)
    gf2 = jnp.stack([g1, g2], axis=-1).reshape(128, 128)
    imp8 = imp.reshape(N_TOKENS // 1024, NUM_EXPERTS)
    a_sc2, a_cb2, gq2, aux = _dispatch(fi2, gf2, imp8)
    xe = _scgather(a_sc2.reshape(M_SLOTS), x2)
    outbuf = _ffn(xe, W1, b1, W2, b2)
    y = _combine(outbuf, a_cb2.reshape(M_SLOTS), gq2.reshape(M_SLOTS))
    return y.reshape(B, T, C), aux[0, 0]


# speedup vs baseline: 2.2762x; 1.5590x over previous
import jax
import jax.numpy as jnp
from jax import lax
from jax.experimental import pallas as pl
from jax.experimental.pallas import tpu as pltpu
from jax.experimental.pallas import tpu_sc as plsc

D_MODEL = 1024
NUM_EXPERTS = 8
HIDDEN = 4096
TOP_K = 2
N_TOKENS = 8192
M_SLOTS = N_TOKENS * TOP_K
CAP = 1280
EC = NUM_EXPERTS * CAP

NW = 32
ROWS_PER_W = EC // NW
GCHUNK = 64
TOK_PER_W = N_TOKENS // NW
CCHUNK = 16

RB = 256
HB = 512
N_RB = CAP // RB
N_HB = HIDDEN // HB

_INV_SQRT2 = 0.7071067811865476


def _router_body(x_ref, wr_ref, br_ref, e1_ref, e2_ref, g1_ref, g2_ref, imp_ref):
    xb = x_ref[...]
    logits = jnp.dot(xb, wr_ref[...], preferred_element_type=jnp.float32)
    logits = logits + br_ref[0, :][None, :]
    mx = jnp.max(logits, axis=-1, keepdims=True)
    p = jnp.exp(logits - mx)
    s = jnp.sum(p, axis=-1, keepdims=True)
    probs = p / s
    imp_ref[0, 0, :] = jnp.sum(probs, axis=0)

    iota8 = lax.broadcasted_iota(jnp.int32, probs.shape, 1)
    m1 = jnp.max(probs, axis=-1, keepdims=True)
    i1 = jnp.min(jnp.where(probs == m1, iota8, NUM_EXPERTS), axis=-1)
    mask1 = iota8 == i1[:, None]
    probs2 = jnp.where(mask1, -jnp.inf, probs)
    m2 = jnp.max(probs2, axis=-1, keepdims=True)
    i2 = jnp.min(jnp.where(probs2 == m2, iota8, NUM_EXPERTS), axis=-1)
    gs = m1[:, 0] + m2[:, 0]
    e1_ref[...] = i1
    e2_ref[...] = i2
    g1_ref[...] = m1[:, 0] / gs
    g2_ref[...] = m2[:, 0] / gs


def _router(x2, Wr, br2):
    nblk = N_TOKENS // 1024
    return pl.pallas_call(
        _router_body,
        grid=(nblk,),
        in_specs=[
            pl.BlockSpec((1024, D_MODEL), lambda b: (b, 0)),
            pl.BlockSpec((D_MODEL, NUM_EXPERTS), lambda b: (0, 0)),
            pl.BlockSpec((1, NUM_EXPERTS), lambda b: (0, 0)),
        ],
        out_specs=[
            pl.BlockSpec((1024,), lambda b: (b,)),
            pl.BlockSpec((1024,), lambda b: (b,)),
            pl.BlockSpec((1024,), lambda b: (b,)),
            pl.BlockSpec((1024,), lambda b: (b,)),
            pl.BlockSpec((1, 1, NUM_EXPERTS), lambda b: (b, 0, 0)),
        ],
        out_shape=[
            jax.ShapeDtypeStruct((N_TOKENS,), jnp.int32),
            jax.ShapeDtypeStruct((N_TOKENS,), jnp.int32),
            jax.ShapeDtypeStruct((N_TOKENS,), jnp.float32),
            jax.ShapeDtypeStruct((N_TOKENS,), jnp.float32),
            jax.ShapeDtypeStruct((nblk, 1, NUM_EXPERTS), jnp.float32),
        ],
    )(x2, Wr, br2)


def _dispatch_body(fi_ref, gf_ref, imp_ref, asc_ref, acb_ref, gq_ref, aux_ref):
    fi = fi_ref[...]
    gf = gf_ref[...]
    r_i = lax.broadcasted_iota(jnp.int32, (128, 128), 0)
    c_i = lax.broadcasted_iota(jnp.int32, (128, 128), 1)
    U = (r_i < c_i).astype(jnp.float32)
    L = (r_i > c_i).astype(jnp.float32)

    rank = jnp.zeros((128, 128), jnp.float32)
    totals = []
    for e in range(NUM_EXPERTS):
        me = (fi == e).astype(jnp.float32)
        pe = jnp.dot(me, U, preferred_element_type=jnp.float32)
        se = jnp.sum(me, axis=-1, keepdims=True)
        oe = jnp.dot(L, se, preferred_element_type=jnp.float32)
        rank = rank + (pe + oe) * me
        totals.append(jnp.sum(se))
    rank = rank.astype(jnp.int32)
    valid = rank < CAP
    addr = fi * CAP + rank
    asc_ref[...] = jnp.where(valid, addr, jnp.int32(1 << 20))
    acb_ref[...] = fi * CAP + lax.rem(rank, CAP)
    gq_ref[...] = jnp.where(valid, gf, 0.0)

    imp = jnp.sum(imp_ref[...], axis=0)
    load = jnp.minimum(jnp.stack(totals), float(CAP))
    impn = imp / jnp.sum(imp)
    loadn = load / jnp.sum(load)
    aux = jnp.sum(impn * loadn) * float(NUM_EXPERTS * NUM_EXPERTS)
    aux_ref[...] = jnp.broadcast_to(aux, (1, 1))


def _dispatch(fi2, gf2, imp8):
    return pl.pallas_call(
        _dispatch_body,
        out_shape=[
            jax.ShapeDtypeStruct((128, 128), jnp.int32),
            jax.ShapeDtypeStruct((128, 128), jnp.int32),
            jax.ShapeDtypeStruct((128, 128), jnp.float32),
            jax.ShapeDtypeStruct((1, 1), jnp.float32),
        ],
    )(fi2, gf2, imp8)


def _sc_mesh():
    return plsc.VectorSubcoreMesh(core_axis_name="c", subcore_axis_name="s")


def _scgather_body(asc_hbm, x_hbm, xe_hbm, asc_v, seg_v, rows_v, sem):
    wid = lax.axis_index("s") * 2 + lax.axis_index("c")
    lo = wid * ROWS_PER_W
    pltpu.sync_copy(asc_hbm, asc_v)
    zero = jnp.zeros((16,), jnp.int32)
    for j in range(ROWS_PER_W // 16):
        seg_v[pl.ds(j * 16, 16)] = zero

    iota = lax.iota(jnp.int32, 16)

    def step(i, carry):
        a = asc_v[pl.ds(i * 16, 16)]
        la = a - lo
        mask = (a >= lo) & (a < lo + ROWS_PER_W)
        val = lax.shift_right_logical(i * 16 + iota, 1)
        plsc.store_scatter(seg_v, [jnp.where(mask, la, 0)], val, mask=mask)
        return carry

    lax.fori_loop(0, M_SLOTS // 16, step, 0)

    for ch in range(ROWS_PER_W // GCHUNK):
        pltpu.async_copy(
            x_hbm.at[seg_v.at[pl.ds(ch * GCHUNK, GCHUNK)]], rows_v, sem
        ).wait()
        pltpu.sync_copy(rows_v, xe_hbm.at[pl.ds(lo + ch * GCHUNK, GCHUNK)])


def _scgather(a_sc, x2):
    k = pl.kernel(
        _scgather_body,
        out_type=jax.ShapeDtypeStruct((EC, D_MODEL), jnp.float32),
        mesh=_sc_mesh(),
        compiler_params=pltpu.CompilerParams(needs_layout_passes=False),
        scratch_types=[
            pltpu.VMEM((M_SLOTS,), jnp.int32),
            pltpu.VMEM((ROWS_PER_W,), jnp.int32),
            pltpu.VMEM((GCHUNK, D_MODEL), jnp.float32),
            pltpu.SemaphoreType.DMA,
        ],
    )
    return k(a_sc, x2)


def _ffn_body(x_ref, w1_ref, b1_ref, w2_ref, b2_ref, o_ref):
    h = pl.program_id(1)
    xb = x_ref[...].astype(jnp.bfloat16)
    hb = jnp.dot(xb, w1_ref[0].astype(jnp.bfloat16),
                 preferred_element_type=jnp.float32) + b1_ref[0]
    hb = 0.5 * hb * (1.0 + lax.erf(hb * _INV_SQRT2))
    contrib = jnp.dot(hb.astype(jnp.bfloat16), w2_ref[0].astype(jnp.bfloat16),
                      preferred_element_type=jnp.float32)

    @pl.when(h == 0)
    def _():
        o_ref[...] = contrib + b2_ref[0]

    @pl.when(h > 0)
    def _():
        o_ref[...] = o_ref[...] + contrib


def _ffn(xe, W1, b1, W2, b2):
    return pl.pallas_call(
        _ffn_body,
        grid=(NUM_EXPERTS, N_HB),
        in_specs=[
            pl.BlockSpec((CAP, D_MODEL), lambda e, h: (e, 0)),
            pl.BlockSpec((1, D_MODEL, HB), lambda e, h: (e, 0, h)),
            pl.BlockSpec((1, 1, HB), lambda e, h: (e, 0, h)),
            pl.BlockSpec((1, HB, D_MODEL), lambda e, h: (e, h, 0)),
            pl.BlockSpec((1, 1, D_MODEL), lambda e, h: (e, 0, 0)),
        ],
        out_specs=pl.BlockSpec((CAP, D_MODEL), lambda e, h: (e, 0)),
        out_shape=jax.ShapeDtypeStruct((EC, D_MODEL), jnp.float32),
        compiler_params=pltpu.CompilerParams(
            dimension_semantics=("parallel", "arbitrary"),
        ),
    )(xe, W1, b1.reshape(NUM_EXPERTS, 1, HIDDEN), W2,
      b2.reshape(NUM_EXPERTS, 1, D_MODEL))


def _combine_body(ob_hbm, acb_hbm, gq_hbm, y_hbm, acb_v, gq_v, idx_v,
                  rows_v, y_v, gsem, osem):
    wid = lax.axis_index("s") * 2 + lax.axis_index("c")
    tbase = wid * TOK_PER_W
    sbase = tbase * 2
    pltpu.sync_copy(acb_hbm.at[pl.ds(sbase, TOK_PER_W * 2)], acb_v)
    pltpu.sync_copy(gq_hbm.at[pl.ds(sbase, TOK_PER_W * 2)], gq_v)
    iota = lax.iota(jnp.int32, 16)
    nch = TOK_PER_W // CCHUNK

    def fire(c, p):
        sidx = 2 * (c * CCHUNK + iota)
        idx_v[pl.ds(p * 2 * CCHUNK, 16)] = plsc.load_gather(acb_v, [sidx])
        idx_v[pl.ds(p * 2 * CCHUNK + 16, 16)] = plsc.load_gather(acb_v, [sidx + 1])
        return pltpu.async_copy(
            ob_hbm.at[idx_v.at[pl.ds(p * 2 * CCHUNK, 2 * CCHUNK)]],
            rows_v.at[p], gsem,
        )

    cps = {0: fire(0, 0)}
    ocs = {}
    for c in range(nch):
        p = c & 1
        if c + 1 < nch:
            cps[(c + 1) & 1] = fire(c + 1, (c + 1) & 1)
        cps[p].wait()
        if c >= 2:
            ocs[p].wait()

        def tok(t, carry):
            st = 2 * (c * CCHUNK + t)
            ga = plsc.load_gather(gq_v, [jnp.full((16,), st, jnp.int32)])
            gb = plsc.load_gather(gq_v, [jnp.full((16,), st + 1, jnp.int32)])
            for j in range(D_MODEL // 16):
                y_v[p, t, pl.ds(j * 16, 16)] = (
                    ga * rows_v[p, t, pl.ds(j * 16, 16)]
                    + gb * rows_v[p, CCHUNK + t, pl.ds(j * 16, 16)]
                )
            return carry

        lax.fori_loop(0, CCHUNK, tok, 0)
        ocs[p] = pltpu.async_copy(
            y_v.at[p], y_hbm.at[pl.ds(tbase + c * CCHUNK, CCHUNK)], osem
        )
    ocs[(nch - 2) & 1].wait()
    ocs[(nch - 1) & 1].wait()


def _combine(outbuf, a_cb, gq):
    k = pl.kernel(
        _combine_body,
        out_type=jax.ShapeDtypeStruct((N_TOKENS, D_MODEL), jnp.float32),
        mesh=_sc_mesh(),
        compiler_params=pltpu.CompilerParams(needs_layout_passes=False),
        scratch_types=[
            pltpu.VMEM((TOK_PER_W * 2,), jnp.int32),
            pltpu.VMEM((TOK_PER_W * 2,), jnp.float32),
            pltpu.VMEM((2 * 2 * CCHUNK,), jnp.int32),
            pltpu.VMEM((2, 2 * CCHUNK, D_MODEL), jnp.float32),
            pltpu.VMEM((2, CCHUNK, D_MODEL), jnp.float32),
            pltpu.SemaphoreType.DMA,
            pltpu.SemaphoreType.DMA,
        ],
    )
    return k(outbuf, a_cb, gq)


def kernel(x, Wr, br, W1, b1, W2, b2):
    B, T, C = x.shape
    x2 = x.reshape(B * T, C)
    e1, e2, g1, g2, imp = _router(x2, Wr, br.reshape(1, NUM_EXPERTS))
    fi2 = jnp.stack([e1, e2], axis=-1).reshape(128, 128)
    gf2 = jnp.stack([g1, g2], axis=-1).reshape(128, 128)
    imp8 = imp.reshape(N_TOKENS // 1024, NUM_EXPERTS)
    a_sc2, a_cb2, gq2, aux = _dispatch(fi2, gf2, imp8)
    xe = _scgather(a_sc2.reshape(M_SLOTS), x2)
    outbuf = _ffn(xe, W1, b1, W2, b2)
    y = _combine(outbuf, a_cb2.reshape(M_SLOTS), gq2.reshape(M_SLOTS))
    return y.reshape(B, T, C), aux[0, 0]
